# baseline (speedup 1.0000x reference)
import functools

import jax
import jax.numpy as jnp
from jax import lax
from jax.experimental import pallas as pl
from jax.experimental.pallas import tpu as pltpu
from jax.experimental.pallas import tpu_sc as plsc


@functools.lru_cache(maxsize=None)
def _build_idx(B, H, W):
    info = plsc.get_sparse_core_info()
    NC, NS, L = info.num_cores, info.num_subcores, info.num_lanes
    NW = NC * NS
    assert B % NW == 0
    bpw = B // NW
    C = 8192
    assert bpw % C == 0
    n_chunks = bpw // C
    NB = 2

    mesh = plsc.VectorSubcoreMesh(core_axis_name="c", subcore_axis_name="s")

    scratch = {}
    for b in range(NB):
        scratch[f"zx{b}"] = pltpu.VMEM((C,), jnp.float32)
        scratch[f"zy{b}"] = pltpu.VMEM((C,), jnp.float32)
        scratch[f"idx{b}"] = pltpu.VMEM((C,), jnp.int32)
        scratch[f"sem_z{b}"] = pltpu.SemaphoreType.DMA
        scratch[f"sem_o{b}"] = pltpu.SemaphoreType.DMA
    scratch["pv"] = pltpu.VMEM((4, L), jnp.float32)
    scratch["sem_p"] = pltpu.SemaphoreType.DMA

    @functools.partial(
        pl.kernel,
        mesh=mesh,
        out_type=jax.ShapeDtypeStruct((B,), jnp.int32),
        scratch_types=scratch,
    )
    def k(zx_hbm, zy_hbm, p_hbm, idx_hbm, **s):
        wid = lax.axis_index("s") * NC + lax.axis_index("c")
        base = wid * bpw
        pltpu.async_copy(p_hbm, s["pv"], s["sem_p"]).wait()
        shift_x = s["pv"][0]
        shift_y = s["pv"][1]
        scale_x = s["pv"][2]
        scale_y = s["pv"][3]
        szx = jnp.float32(H - 1)
        szy = jnp.float32(W - 1)

        def start_z(c):
            cb = base + c * C
            pltpu.async_copy(zx_hbm.at[pl.ds(cb, C)], s[f"zx{c % NB}"], s[f"sem_z{c % NB}"])
            pltpu.async_copy(zy_hbm.at[pl.ds(cb, C)], s[f"zy{c % NB}"], s[f"sem_z{c % NB}"])

        def wait_z(c):
            cb = base + c * C
            pltpu.make_async_copy(zx_hbm.at[pl.ds(cb, C)], s[f"zx{c % NB}"], s[f"sem_z{c % NB}"]).wait()
            pltpu.make_async_copy(zy_hbm.at[pl.ds(cb, C)], s[f"zy{c % NB}"], s[f"sem_z{c % NB}"]).wait()

        def compute(c):
            zx, zy, idx = s[f"zx{c % NB}"], s[f"zy{c % NB}"], s[f"idx{c % NB}"]

            def vec_body(j, carry):
                vx = zx[pl.ds(j * L, L)]
                vy = zy[pl.ds(j * L, L)]
                tx = jnp.clip((vx - shift_x) / scale_x, 0.0, 1.0)
                ty = jnp.clip((vy - shift_y) / scale_y, 0.0, 1.0)
                ix = (tx * szx).astype(jnp.int32)
                iy = (ty * szy).astype(jnp.int32)
                idx[pl.ds(j * L, L)] = ix * W + iy
                return carry

            lax.fori_loop(0, C // L, vec_body, 0, unroll=4)

        def start_out(c):
            cb = base + c * C
            pltpu.async_copy(s[f"idx{c % NB}"], idx_hbm.at[pl.ds(cb, C)], s[f"sem_o{c % NB}"])

        def wait_out(c):
            cb = base + c * C
            pltpu.make_async_copy(s[f"idx{c % NB}"], idx_hbm.at[pl.ds(cb, C)], s[f"sem_o{c % NB}"]).wait()

        start_z(0)
        for c in range(n_chunks):
            if c + 1 < n_chunks:
                start_z(c + 1)
            wait_z(c)
            if c >= NB:
                wait_out(c - NB)
            compute(c)
            start_out(c)
        for c in range(max(0, n_chunks - NB), n_chunks):
            wait_out(c)

    return k


@functools.lru_cache(maxsize=None)
def _build_gather(B, HW):
    info = plsc.get_sparse_core_info()
    NC, NS, L = info.num_cores, info.num_subcores, info.num_lanes
    NW = NC * NS
    assert B % NW == 0
    bpw = B // NW
    C = 4096
    assert bpw % C == 0
    n_chunks = bpw // C
    NB = 4

    mesh = plsc.VectorSubcoreMesh(core_axis_name="c", subcore_axis_name="s")

    scratch = {}
    for b in range(NB):
        scratch[f"idx{b}"] = pltpu.VMEM((C,), jnp.int32)
        scratch[f"val{b}"] = pltpu.VMEM((C,), jnp.float32)
        scratch[f"sem_i{b}"] = pltpu.SemaphoreType.DMA
        scratch[f"sem_g{b}"] = pltpu.SemaphoreType.DMA
        scratch[f"sem_o{b}"] = pltpu.SemaphoreType.DMA

    @functools.partial(
        pl.kernel,
        mesh=mesh,
        out_type=jax.ShapeDtypeStruct((B,), jnp.float32),
        scratch_types=scratch,
    )
    def k(idx_hbm, d_hbm, out_hbm, **s):
        wid = lax.axis_index("s") * NC + lax.axis_index("c")
        base = wid * bpw

        def start_idx(c):
            cb = base + c * C
            pltpu.async_copy(idx_hbm.at[pl.ds(cb, C)], s[f"idx{c % NB}"], s[f"sem_i{c % NB}"])

        def wait_idx(c):
            cb = base + c * C
            pltpu.make_async_copy(idx_hbm.at[pl.ds(cb, C)], s[f"idx{c % NB}"], s[f"sem_i{c % NB}"]).wait()

        def start_gather(c):
            pltpu.async_copy(d_hbm.at[s[f"idx{c % NB}"]], s[f"val{c % NB}"], s[f"sem_g{c % NB}"])

        def wait_gather(c):
            pltpu.make_async_copy(d_hbm.at[s[f"idx{c % NB}"]], s[f"val{c % NB}"], s[f"sem_g{c % NB}"]).wait()

        def start_out(c):
            cb = base + c * C
            pltpu.async_copy(s[f"val{c % NB}"], out_hbm.at[pl.ds(cb, C)], s[f"sem_o{c % NB}"])

        def wait_out(c):
            cb = base + c * C
            pltpu.make_async_copy(s[f"val{c % NB}"], out_hbm.at[pl.ds(cb, C)], s[f"sem_o{c % NB}"]).wait()

        start_idx(0)
        start_idx(1)
        wait_idx(0)
        start_gather(0)
        wait_idx(1)
        start_gather(1)
        for c in range(2, n_chunks):
            start_idx(c)
            wait_gather(c - 2)
            start_out(c - 2)
            if c >= NB:
                wait_out(c - NB)
            wait_idx(c)
            start_gather(c)
        for c in range(max(0, n_chunks - 2), n_chunks):
            wait_gather(c)
            start_out(c)
        for c in range(max(0, n_chunks - NB), n_chunks):
            wait_out(c)

    return k


def kernel(z, density, scale, shift, image_size):
    B = z.shape[0]
    H, W = density.shape
    L = plsc.get_sparse_core_info().num_lanes
    zx = z[:, 0]
    zy = z[:, 1]
    dflat = density.reshape(-1)
    params = jnp.concatenate(
        [
            jnp.broadcast_to(shift.reshape(2, 1), (2, L)),
            jnp.broadcast_to(scale.reshape(2, 1), (2, L)),
        ],
        axis=0,
    ).astype(jnp.float32)
    idx = _build_idx(B, H, W)(zx, zy, params)
    return _build_gather(B, H * W)(idx, dflat)


# speedup vs baseline: 1.1969x; 1.1969x over previous
import functools

import jax
import jax.numpy as jnp
from jax import lax
from jax.experimental import pallas as pl
from jax.experimental.pallas import tpu as pltpu
from jax.experimental.pallas import tpu_sc as plsc


@functools.lru_cache(maxsize=None)
def _build_flatten(H, W):
    info = plsc.get_sparse_core_info()
    NC, NS = info.num_cores, info.num_subcores
    NW = NC * NS
    assert H % (8 * NW) == 0
    n_slabs = H // 8
    spw = n_slabs // NW

    mesh = plsc.VectorSubcoreMesh(core_axis_name="c", subcore_axis_name="s")

    @functools.partial(
        pl.kernel,
        mesh=mesh,
        out_type=jax.ShapeDtypeStruct((H * W,), jnp.float32),
        scratch_types=[
            pltpu.VMEM((8, W), jnp.float32),
            pltpu.VMEM((8, W), jnp.float32),
            pltpu.SemaphoreType.DMA,
            pltpu.SemaphoreType.DMA,
            pltpu.SemaphoreType.DMA,
            pltpu.SemaphoreType.DMA,
        ],
    )
    def k(d_hbm, tbl_hbm, buf0, buf1, sr0, sr1, sw0, sw1):
        wid = lax.axis_index("s") * NC + lax.axis_index("c")
        base = wid * spw
        bufs = [(buf0, sr0, sw0), (buf1, sr1, sw1)]

        def start_read(t):
            buf, sr, _ = bufs[t % 2]
            sl = base + t
            pltpu.async_copy(d_hbm.at[pl.ds(sl * 8, 8), :], buf, sr)

        def wait_read(t):
            buf, sr, _ = bufs[t % 2]
            sl = base + t
            pltpu.make_async_copy(d_hbm.at[pl.ds(sl * 8, 8), :], buf, sr).wait()

        def start_writes(t):
            buf, _, sw = bufs[t % 2]
            sl = base + t
            for r in range(8):
                pltpu.async_copy(buf.at[r], tbl_hbm.at[pl.ds((sl * 8 + r) * W, W)], sw)

        def wait_writes(t):
            buf, _, sw = bufs[t % 2]
            sl = base + t
            for r in range(8):
                pltpu.make_async_copy(
                    buf.at[r], tbl_hbm.at[pl.ds((sl * 8 + r) * W, W)], sw
                ).wait()

        start_read(0)
        for t in range(spw):
            if t + 1 < spw:
                start_read(t + 1)
            if t >= 2:
                wait_writes(t - 2)
            wait_read(t)
            start_writes(t)
        for t in range(max(0, spw - 2), spw):
            wait_writes(t)

    return k


@functools.lru_cache(maxsize=None)
def _build_gather(B, H, W):
    info = plsc.get_sparse_core_info()
    NC, NS, L = info.num_cores, info.num_subcores, info.num_lanes
    NW = NC * NS
    assert B % NW == 0
    bpw = B // NW
    C = 4096
    assert bpw % C == 0
    n_chunks = bpw // C

    mesh = plsc.VectorSubcoreMesh(core_axis_name="c", subcore_axis_name="s")

    NB = 4
    scratch = {}
    for b in range(NB):
        scratch[f"zx{b}"] = pltpu.VMEM((C,), jnp.float32)
        scratch[f"zy{b}"] = pltpu.VMEM((C,), jnp.float32)
        scratch[f"idx{b}"] = pltpu.VMEM((C,), jnp.int32)
        scratch[f"val{b}"] = pltpu.VMEM((C,), jnp.float32)
        scratch[f"sem_z{b}"] = pltpu.SemaphoreType.DMA
        scratch[f"sem_g{b}"] = pltpu.SemaphoreType.DMA
        scratch[f"sem_o{b}"] = pltpu.SemaphoreType.DMA
    scratch["pv"] = pltpu.VMEM((4, L), jnp.float32)
    scratch["sem_p"] = pltpu.SemaphoreType.DMA

    @functools.partial(
        pl.kernel,
        mesh=mesh,
        out_type=jax.ShapeDtypeStruct((B,), jnp.float32),
        scratch_types=scratch,
    )
    def k(zx_hbm, zy_hbm, d_hbm, p_hbm, out_hbm, **s):
        wid = lax.axis_index("s") * NC + lax.axis_index("c")
        base = wid * bpw
        pltpu.async_copy(p_hbm, s["pv"], s["sem_p"]).wait()
        shift_x = s["pv"][0]
        shift_y = s["pv"][1]
        scale_x = s["pv"][2]
        scale_y = s["pv"][3]
        szx = jnp.float32(H - 1)
        szy = jnp.float32(W - 1)

        zbufs = [(s[f"zx{b}"], s[f"zy{b}"], s[f"sem_z{b}"]) for b in range(NB)]
        gbufs = [(s[f"idx{b}"], s[f"val{b}"], s[f"sem_g{b}"], s[f"sem_o{b}"])
                 for b in range(NB)]

        def start_z(c):
            zx, zy, sem = zbufs[c % NB]
            cb = base + c * C
            pltpu.async_copy(zx_hbm.at[pl.ds(cb, C)], zx, sem)
            pltpu.async_copy(zy_hbm.at[pl.ds(cb, C)], zy, sem)

        def wait_z(c):
            zx, zy, sem = zbufs[c % NB]
            cb = base + c * C
            pltpu.make_async_copy(zx_hbm.at[pl.ds(cb, C)], zx, sem).wait()
            pltpu.make_async_copy(zy_hbm.at[pl.ds(cb, C)], zy, sem).wait()

        def compute_idx(c):
            zx, zy, _ = zbufs[c % NB]
            idx = gbufs[c % NB][0]

            def vec_body(j, carry):
                vx = zx[pl.ds(j * L, L)]
                vy = zy[pl.ds(j * L, L)]
                tx = jnp.clip((vx - shift_x) / scale_x, 0.0, 1.0)
                ty = jnp.clip((vy - shift_y) / scale_y, 0.0, 1.0)
                ix = (tx * szx).astype(jnp.int32)
                iy = (ty * szy).astype(jnp.int32)
                idx[pl.ds(j * L, L)] = ix * W + iy
                return carry

            lax.fori_loop(0, C // L, vec_body, 0, unroll=4)

        def start_gather(c):
            idx, val, sem, _ = gbufs[c % NB]
            pltpu.async_copy(d_hbm.at[idx], val, sem)

        def wait_gather(c):
            idx, val, sem, _ = gbufs[c % NB]
            pltpu.make_async_copy(d_hbm.at[idx], val, sem).wait()

        def start_out(c):
            _, val, _, sem = gbufs[c % NB]
            cb = base + c * C
            pltpu.async_copy(val, out_hbm.at[pl.ds(cb, C)], sem)

        def wait_out(c):
            _, val, _, sem = gbufs[c % NB]
            cb = base + c * C
            pltpu.make_async_copy(val, out_hbm.at[pl.ds(cb, C)], sem).wait()

        start_z(0)
        start_z(1)
        wait_z(0)
        compute_idx(0)
        start_gather(0)
        wait_z(1)
        compute_idx(1)
        start_gather(1)
        for c in range(2, n_chunks):
            start_z(c)
            wait_z(c)
            compute_idx(c)
            wait_gather(c - 2)
            start_out(c - 2)
            if c >= 4:
                wait_out(c - 4)
            start_gather(c)
        for c in range(max(0, n_chunks - 2), n_chunks):
            wait_gather(c)
            start_out(c)
        for c in range(max(0, n_chunks - 4), n_chunks):
            wait_out(c)

    return k


def kernel(z, density, scale, shift, image_size):
    B = z.shape[0]
    H, W = density.shape
    L = plsc.get_sparse_core_info().num_lanes
    zx = z[:, 0]
    zy = z[:, 1]
    tbl = _build_flatten(H, W)(density)
    params = jnp.concatenate(
        [
            jnp.broadcast_to(shift.reshape(2, 1), (2, L)),
            jnp.broadcast_to(scale.reshape(2, 1), (2, L)),
        ],
        axis=0,
    ).astype(jnp.float32)
    return _build_gather(B, H, W)(zx, zy, tbl, params)
